# baseline (speedup 1.0000x reference)
import functools

import jax
import jax.numpy as jnp
from jax import lax
from jax.experimental import pallas as pl
from jax.experimental.pallas import tpu as pltpu
from jax.experimental.pallas import tpu_sc as plsc

_N = 10000
_E = 320000

_NC = 2
_NS = 16
_L = 16
_NW = _NC * _NS

_S = 128
_RW = 80
_EP = _NW * _RW * _S
_R0 = 128
_R1 = 32
_C0E = _NS * _R0 * _S
_NP = 10112
_RPT = _NP // _NS
_G = _S // _L


@functools.lru_cache(None)
def _make_sc_aggregate(nv: int):
    D = 128
    full, rem = divmod(_RPT, _S)
    mesh = plsc.VectorSubcoreMesh(core_axis_name="c", subcore_axis_name="s")

    @functools.partial(
        pl.kernel,
        mesh=mesh,
        out_type=jax.ShapeDtypeStruct((_NC * _NP, D), jnp.float32),
        scratch_types=[
            pltpu.VMEM((8, _S), jnp.int32),
            pltpu.VMEM((8, _S), jnp.int32),
            pltpu.VMEM((8, _S), jnp.float32),
            pltpu.VMEM((_S, D), jnp.float32),
            pltpu.VMEM((_S, D), jnp.float32),
            pltpu.VMEM_SHARED((_NP, D), jnp.float32),
            pltpu.SemaphoreType.DMA,
            pltpu.SemaphoreType.DMA,
        ],
    )
    def agg_kernel(feat, src3, dst3, ew3, out,
                   srcb, dstb, ewb, g0, g1, accum, sem_g0, sem_g1):
        cid = lax.axis_index("c")
        sid = lax.axis_index("s")
        wid = cid * _NS + sid
        nrows = jnp.where(cid == 0, _R0, _R1)
        base = sid * _RPT
        gbuf = (g0, g1)
        sem_g = (sem_g0, sem_g1)

        def fire_gather(row, b):
            pltpu.async_copy(feat.at[srcb.at[row]], gbuf[b], sem_g[b])

        def wait_gather(b):
            pltpu.make_async_copy(feat.at[pl.ds(0, _S)], gbuf[b],
                                  sem_g[b]).wait()

        def scale(row, b):
            g = gbuf[b]

            def grp(gi, c):
                w = ewb[row, pl.ds(gi * _L, _L)]
                for i in range(_L):
                    e = gi * _L + i
                    sc = w[i]
                    for k in range(nv):
                        sl = pl.ds(k * _L, _L)
                        g[e, sl] = g[e, sl] * sc
                return c

            lax.fori_loop(0, _G, grp, 0, unroll=2)

        def scatter(row, b):
            pltpu.sync_copy(gbuf[b], accum.at[dstb.at[row]], add=True)

        def stage(blk_rows0, dst_rows0):
            pltpu.sync_copy(src3.at[wid, pl.ds(blk_rows0, 4)],
                            srcb.at[pl.ds(dst_rows0, 4)])
            pltpu.sync_copy(dst3.at[wid, pl.ds(blk_rows0, 4)],
                            dstb.at[pl.ds(dst_rows0, 4)])
            pltpu.sync_copy(ew3.at[wid, pl.ds(blk_rows0, 4)],
                            ewb.at[pl.ds(dst_rows0, 4)])

        def zero_row(e, carry):
            for k in range(D // _L):
                g0[e, pl.ds(k * _L, _L)] = jnp.zeros((_L,), jnp.float32)
            return carry

        lax.fori_loop(0, _S, zero_row, 0)
        for j in range(full):
            pltpu.sync_copy(g0, accum.at[pl.ds(base + j * _S, _S)])
        if rem:
            pltpu.sync_copy(g0.at[pl.ds(0, rem)],
                            accum.at[pl.ds(base + full * _S, rem)])
        plsc.subcore_barrier()

        stage(0, 0)
        fire_gather(0, 0)
        fire_gather(1, 1)

        nblk = nrows // 4

        def blk_body(blk, carry):
            bm4 = lax.rem(blk, 2) * 4
            nm4 = lax.rem(blk + 1, 2) * 4

            @pl.when(blk < nblk - 1)
            def _():
                stage((blk + 1) * 4, nm4)

            for jj in range(4):
                b = jj % 2
                row = bm4 + jj
                wait_gather(b)
                scale(row, b)
                scatter(row, b)
                if jj < 2:
                    @pl.when(blk * 4 + jj + 2 < nrows)
                    def _():
                        fire_gather(bm4 + jj + 2, b)
                else:
                    @pl.when(blk * 4 + jj + 2 < nrows)
                    def _():
                        fire_gather(nm4 + jj - 2, b)
            return carry

        lax.fori_loop(0, nblk, blk_body, 0)
        plsc.subcore_barrier()

        def drain(r0, n):
            pltpu.sync_copy(accum.at[pl.ds(base + r0, n)], g0.at[pl.ds(0, n)])
            pltpu.sync_copy(g0.at[pl.ds(0, n)],
                            out.at[pl.ds(cid * _NP + base + r0, n)])

        for j in range(full):
            drain(j * _S, _S)
        if rem:
            drain(full * _S, rem)

    return agg_kernel


def _mm(a, w):
    return lax.dot_general(a, w, (((1,), (1,)), ((), ())),
                           preferred_element_type=jnp.float32)


def _bn_tanh(z, g, b):
    mu = jnp.mean(z, axis=0, keepdims=True)
    d = z - mu
    var = jnp.mean(d * d, axis=0, keepdims=True)
    return jnp.tanh(d * lax.rsqrt(var + 1e-5) * g + b)


def _tc1(aggp, x, w_r, b_r, w_s, g, be):
    def body(aggp_ref, x_ref, wr_ref, br_ref, ws_ref, g_ref, be_ref, o_ref):
        agg = aggp_ref[:_N] + aggp_ref[_NP:_NP + _N]
        z = _mm(agg, wr_ref[...]) + br_ref[...] + _mm(x_ref[...], ws_ref[...])
        o_ref[...] = _bn_tanh(z, g_ref[...], be_ref[...])

    h = w_r.shape[0]
    return pl.pallas_call(
        body, out_shape=jax.ShapeDtypeStruct((_N, h), jnp.float32),
    )(aggp, x, w_r, b_r, w_s, g, be)


def _tc2(aggp, h1, w_r, b_r, w_s, g, be, w3r):
    def body(aggp_ref, h1_ref, wr_ref, br_ref, ws_ref, g_ref, be_ref,
             w3r_ref, h2_ref, p3_ref):
        agg = aggp_ref[:_N] + aggp_ref[_NP:_NP + _N]
        z = _mm(agg, wr_ref[...]) + br_ref[...] + _mm(h1_ref[...], ws_ref[...])
        h2 = _bn_tanh(z, g_ref[...], be_ref[...])
        h2_ref[...] = h2
        p3 = _mm(h2, w3r_ref[...])
        p3_ref[...] = jnp.concatenate(
            [p3, jnp.zeros((_N, 128 - p3.shape[1]), jnp.float32)], axis=1)

    h = w_r.shape[0]
    return pl.pallas_call(
        body,
        out_shape=(jax.ShapeDtypeStruct((_N, h), jnp.float32),
                   jax.ShapeDtypeStruct((_N, 128), jnp.float32)),
    )(aggp, h1, w_r, b_r, w_s, g, be, w3r)


def _tc3(aggp, h2, b_r, w_s, g, be):
    h = w_s.shape[0]

    def body(aggp_ref, h2_ref, br_ref, ws_ref, g_ref, be_ref, o_ref):
        agg = aggp_ref[:_N, :h] + aggp_ref[_NP:_NP + _N, :h]
        z = agg + br_ref[...] + _mm(h2_ref[...], ws_ref[...])
        h3 = _bn_tanh(z, g_ref[...], be_ref[...])
        o_ref[...] = jnp.concatenate(
            [h3, jnp.zeros((_N, 128 - h), jnp.float32)], axis=1)

    return pl.pallas_call(
        body, out_shape=jax.ShapeDtypeStruct((_N, 128), jnp.float32),
    )(aggp, h2, b_r, w_s, g, be)


def _tc4(aggp, h3, w_r, b_r, w_s):
    d = w_r.shape[1]

    def body(aggp_ref, h3_ref, wr_ref, br_ref, ws_ref, o_ref):
        agg = aggp_ref[:_N, :d] + aggp_ref[_NP:_NP + _N, :d]
        o_ref[...] = (_mm(agg, wr_ref[...]) + br_ref[...]
                      + _mm(h3_ref[..., :d], ws_ref[...]))

    h = w_r.shape[0]
    return pl.pallas_call(
        body, out_shape=jax.ShapeDtypeStruct((_N, h), jnp.float32),
    )(aggp, h3, w_r, b_r, w_s)


def kernel(x, edge_index, edge_attr, W1r, b1r, W1s, g1, be1,
           W2r, b2r, W2s, g2, be2, W3r, b3r, W3s, g3, be3, W4r, b4r, W4s):
    pad = _EP - _E

    def repack(a):
        ap = jnp.concatenate([a, jnp.zeros((pad,), a.dtype)])
        p0 = ap[:_C0E].reshape(_NS, _R0, _S)
        p1 = ap[_C0E:].reshape(_NS, _R1, _S)
        p1 = jnp.concatenate(
            [p1, jnp.zeros((_NS, _R0 - _R1, _S), a.dtype)], axis=1)
        return jnp.concatenate([p0, p1], axis=0)

    src3 = repack(edge_index[0])
    dst3 = repack(edge_index[1])
    ew3 = repack(edge_attr)

    agg_w = _make_sc_aggregate(8)
    agg_n = _make_sc_aggregate(4)

    a1 = agg_w(x, src3, dst3, ew3)
    h1 = _tc1(a1, x, W1r, b1r, W1s, g1, be1)
    a2 = agg_w(h1, src3, dst3, ew3)
    h2, p3 = _tc2(a2, h1, W2r, b2r, W2s, g2, be2, W3r)
    a3 = agg_n(p3, src3, dst3, ew3)
    h3 = _tc3(a3, h2, b3r, W3s, g3, be3)
    a4 = agg_n(h3, src3, dst3, ew3)
    return _tc4(a4, h3, W4r, b4r, W4s)


# speedup vs baseline: 1.1847x; 1.1847x over previous
import functools

import jax
import jax.numpy as jnp
from jax import lax
from jax.experimental import pallas as pl
from jax.experimental.pallas import tpu as pltpu
from jax.experimental.pallas import tpu_sc as plsc

_N = 10000
_E = 320000

_NC = 2
_NS = 16
_L = 16
_NW = _NC * _NS

_S = 128
_RW = 80
_EP = _NW * _RW * _S
_R0 = 144
_R1 = 16
_C0E = _NS * _R0 * _S
_NP = 10112
_RPT = _NP // _NS
_G = _S // _L


@functools.lru_cache(None)
def _make_sc_aggregate(nv: int):
    D = 128
    full, rem = divmod(_RPT, _S)
    mesh = plsc.VectorSubcoreMesh(core_axis_name="c", subcore_axis_name="s")

    @functools.partial(
        pl.kernel,
        mesh=mesh,
        out_type=jax.ShapeDtypeStruct((_NC * _NP, D), jnp.float32),
        scratch_types=[
            pltpu.VMEM((8, _S), jnp.int32),
            pltpu.VMEM((8, _S), jnp.int32),
            pltpu.VMEM((8, _S), jnp.float32),
            pltpu.VMEM((_S, D), jnp.float32),
            pltpu.VMEM((_S, D), jnp.float32),
            pltpu.VMEM_SHARED((_NP, D), jnp.float32),
            pltpu.SemaphoreType.DMA,
            pltpu.SemaphoreType.DMA,
        ],
    )
    def agg_kernel(feat, src3, dst3, ew3, out,
                   srcb, dstb, ewb, g0, g1, accum, sem_g0, sem_g1):
        cid = lax.axis_index("c")
        sid = lax.axis_index("s")
        wid = cid * _NS + sid
        nrows = jnp.where(cid == 0, _R0, _R1)
        base = sid * _RPT
        gbuf = (g0, g1)
        sem_g = (sem_g0, sem_g1)

        def fire_gather(row, b):
            pltpu.async_copy(feat.at[srcb.at[row]], gbuf[b], sem_g[b])

        def wait_gather(b):
            pltpu.make_async_copy(feat.at[pl.ds(0, _S)], gbuf[b],
                                  sem_g[b]).wait()

        def scale(row, b):
            g = gbuf[b]

            def grp(gi, c):
                w = ewb[row, pl.ds(gi * _L, _L)]
                for i in range(_L):
                    e = gi * _L + i
                    sc = w[i]
                    for k in range(nv):
                        sl = pl.ds(k * _L, _L)
                        g[e, sl] = g[e, sl] * sc
                return c

            lax.fori_loop(0, _G, grp, 0, unroll=2)

        def scatter(row, b):
            pltpu.sync_copy(gbuf[b], accum.at[dstb.at[row]], add=True)

        def stage(blk_rows0, dst_rows0):
            pltpu.sync_copy(src3.at[wid, pl.ds(blk_rows0, 4)],
                            srcb.at[pl.ds(dst_rows0, 4)])
            pltpu.sync_copy(dst3.at[wid, pl.ds(blk_rows0, 4)],
                            dstb.at[pl.ds(dst_rows0, 4)])
            pltpu.sync_copy(ew3.at[wid, pl.ds(blk_rows0, 4)],
                            ewb.at[pl.ds(dst_rows0, 4)])

        def zero_row(e, carry):
            for k in range(D // _L):
                g0[e, pl.ds(k * _L, _L)] = jnp.zeros((_L,), jnp.float32)
            return carry

        lax.fori_loop(0, _S, zero_row, 0)
        for j in range(full):
            pltpu.sync_copy(g0, accum.at[pl.ds(base + j * _S, _S)])
        if rem:
            pltpu.sync_copy(g0.at[pl.ds(0, rem)],
                            accum.at[pl.ds(base + full * _S, rem)])
        plsc.subcore_barrier()

        stage(0, 0)
        fire_gather(0, 0)
        fire_gather(1, 1)

        nblk = nrows // 4

        def blk_body(blk, carry):
            bm4 = lax.rem(blk, 2) * 4
            nm4 = lax.rem(blk + 1, 2) * 4

            @pl.when(blk < nblk - 1)
            def _():
                stage((blk + 1) * 4, nm4)

            for jj in range(4):
                b = jj % 2
                row = bm4 + jj
                wait_gather(b)
                scale(row, b)
                scatter(row, b)
                if jj < 2:
                    @pl.when(blk * 4 + jj + 2 < nrows)
                    def _():
                        fire_gather(bm4 + jj + 2, b)
                else:
                    @pl.when(blk * 4 + jj + 2 < nrows)
                    def _():
                        fire_gather(nm4 + jj - 2, b)
            return carry

        lax.fori_loop(0, nblk, blk_body, 0)
        plsc.subcore_barrier()

        def drain(r0, n):
            pltpu.sync_copy(accum.at[pl.ds(base + r0, n)], g0.at[pl.ds(0, n)])
            pltpu.sync_copy(g0.at[pl.ds(0, n)],
                            out.at[pl.ds(cid * _NP + base + r0, n)])

        for j in range(full):
            drain(j * _S, _S)
        if rem:
            drain(full * _S, rem)

    return agg_kernel


def _mm(a, w):
    return lax.dot_general(a, w, (((1,), (1,)), ((), ())),
                           preferred_element_type=jnp.float32)


def _bn_tanh(z, g, b):
    mu = jnp.mean(z, axis=0, keepdims=True)
    d = z - mu
    var = jnp.mean(d * d, axis=0, keepdims=True)
    return jnp.tanh(d * lax.rsqrt(var + 1e-5) * g + b)


def _tc1(aggp, x, w_r, b_r, w_s, g, be):
    def body(aggp_ref, x_ref, wr_ref, br_ref, ws_ref, g_ref, be_ref, o_ref):
        agg = aggp_ref[:_N] + aggp_ref[_NP:_NP + _N]
        z = _mm(agg, wr_ref[...]) + br_ref[...] + _mm(x_ref[...], ws_ref[...])
        o_ref[...] = _bn_tanh(z, g_ref[...], be_ref[...])

    h = w_r.shape[0]
    return pl.pallas_call(
        body, out_shape=jax.ShapeDtypeStruct((_N, h), jnp.float32),
    )(aggp, x, w_r, b_r, w_s, g, be)


def _tc2(aggp, h1, w_r, b_r, w_s, g, be, w3r):
    def body(aggp_ref, h1_ref, wr_ref, br_ref, ws_ref, g_ref, be_ref,
             w3r_ref, h2_ref, p3_ref):
        agg = aggp_ref[:_N] + aggp_ref[_NP:_NP + _N]
        z = _mm(agg, wr_ref[...]) + br_ref[...] + _mm(h1_ref[...], ws_ref[...])
        h2 = _bn_tanh(z, g_ref[...], be_ref[...])
        h2_ref[...] = h2
        p3 = _mm(h2, w3r_ref[...])
        p3_ref[...] = jnp.concatenate(
            [p3, jnp.zeros((_N, 128 - p3.shape[1]), jnp.float32)], axis=1)

    h = w_r.shape[0]
    return pl.pallas_call(
        body,
        out_shape=(jax.ShapeDtypeStruct((_N, h), jnp.float32),
                   jax.ShapeDtypeStruct((_N, 128), jnp.float32)),
    )(aggp, h1, w_r, b_r, w_s, g, be, w3r)


def _tc3(aggp, h2, b_r, w_s, g, be):
    h = w_s.shape[0]

    def body(aggp_ref, h2_ref, br_ref, ws_ref, g_ref, be_ref, o_ref):
        agg = aggp_ref[:_N, :h] + aggp_ref[_NP:_NP + _N, :h]
        z = agg + br_ref[...] + _mm(h2_ref[...], ws_ref[...])
        h3 = _bn_tanh(z, g_ref[...], be_ref[...])
        o_ref[...] = jnp.concatenate(
            [h3, jnp.zeros((_N, 128 - h), jnp.float32)], axis=1)

    return pl.pallas_call(
        body, out_shape=jax.ShapeDtypeStruct((_N, 128), jnp.float32),
    )(aggp, h2, b_r, w_s, g, be)


def _tc4(aggp, h3, w_r, b_r, w_s):
    d = w_r.shape[1]

    def body(aggp_ref, h3_ref, wr_ref, br_ref, ws_ref, o_ref):
        agg = aggp_ref[:_N, :d] + aggp_ref[_NP:_NP + _N, :d]
        o_ref[...] = (_mm(agg, wr_ref[...]) + br_ref[...]
                      + _mm(h3_ref[..., :d], ws_ref[...]))

    h = w_r.shape[0]
    return pl.pallas_call(
        body, out_shape=jax.ShapeDtypeStruct((_N, h), jnp.float32),
    )(aggp, h3, w_r, b_r, w_s)


def kernel(x, edge_index, edge_attr, W1r, b1r, W1s, g1, be1,
           W2r, b2r, W2s, g2, be2, W3r, b3r, W3s, g3, be3, W4r, b4r, W4s):
    pad = _EP - _E

    def repack(a):
        ap = jnp.concatenate([a, jnp.zeros((pad,), a.dtype)])
        p0 = ap[:_C0E].reshape(_NS, _R0, _S)
        p1 = ap[_C0E:].reshape(_NS, _R1, _S)
        p1 = jnp.concatenate(
            [p1, jnp.zeros((_NS, _R0 - _R1, _S), a.dtype)], axis=1)
        return jnp.concatenate([p0, p1], axis=0)

    src3 = repack(edge_index[0])
    dst3 = repack(edge_index[1])
    ew3 = repack(edge_attr)

    agg_w = _make_sc_aggregate(8)
    agg_n = _make_sc_aggregate(4)

    a1 = agg_w(x, src3, dst3, ew3)
    h1 = _tc1(a1, x, W1r, b1r, W1s, g1, be1)
    a2 = agg_w(h1, src3, dst3, ew3)
    h2, p3 = _tc2(a2, h1, W2r, b2r, W2s, g2, be2, W3r)
    a3 = agg_n(p3, src3, dst3, ew3)
    h3 = _tc3(a3, h2, b3r, W3s, g3, be3)
    a4 = agg_n(h3, src3, dst3, ew3)
    return _tc4(a4, h3, W4r, b4r, W4s)
